# baseline (speedup 1.0000x reference)
import functools

import jax
import jax.numpy as jnp
from jax import lax
from jax.experimental import pallas as pl
from jax.experimental.pallas import tpu as pltpu
from jax.experimental.pallas import tpu_sc as plsc

NF = 9
H = 128
P = 4096
B = 1024
T = 50

_PBLK = 1024
_SCHUNK = 512
_BBLK = 512
_UNROLL = 5


def _gelu(x):
    return x * (jax.lax.erf(x / 2.0 ** 0.5) + 1) / 2


def _vsn_kernel(x_ref, w1_ref, b1_ref, w2_ref, b2_ref, out_ref):
    x = x_ref[...]
    r = jax.lax.broadcasted_iota(jnp.int32, (T * NF, NF), 0)
    f = jax.lax.broadcasted_iota(jnp.int32, (T * NF, NF), 1)
    sel = (r % NF == f).astype(jnp.float32)
    xm = jnp.dot(x, sel, preferred_element_type=jnp.float32,
                 precision=jax.lax.Precision.HIGHEST) / T
    h = _gelu(jnp.dot(xm, w1_ref[...], preferred_element_type=jnp.float32)
              + b1_ref[...])
    s = jnp.dot(h, w2_ref[...], preferred_element_type=jnp.float32) + b2_ref[...]
    s = s - jnp.max(s, axis=-1, keepdims=True)
    e = jnp.exp(s)
    w = e / jnp.sum(e, axis=-1, keepdims=True)
    rT = jax.lax.broadcasted_iota(jnp.int32, (NF, T * NF), 1)
    fT = jax.lax.broadcasted_iota(jnp.int32, (NF, T * NF), 0)
    selT = (rT % NF == fT).astype(jnp.float32)
    wrep = jnp.dot(w, selT, preferred_element_type=jnp.float32,
                   precision=jax.lax.Precision.HIGHEST)
    out_ref[...] = x * wrep


def _lstm_kernel(xw_ref, pw_ref, pb_ref, wi1_ref, bi1_ref, wh1_ref, bh1_ref,
                 wi2_ref, bi2_ref, wh2_ref, bh2_ref,
                 g_ref, b_ref, te_ref, h1, c1, h2, c2):
    t = pl.program_id(0)

    @pl.when(t == 0)
    def _():
        z = jnp.zeros((B, H), jnp.float32)
        h1[...] = z
        c1[...] = z
        h2[...] = z
        c2[...] = z

    def cell(inp, h, c, wi, bi, wh, bh):
        gates = (jnp.dot(inp, wi, preferred_element_type=jnp.float32) + bi
                 + jnp.dot(h, wh, preferred_element_type=jnp.float32) + bh)
        gi = gates[:, 0 * H:1 * H]
        gf = gates[:, 1 * H:2 * H]
        gg = gates[:, 2 * H:3 * H]
        go = gates[:, 3 * H:4 * H]
        cn = jax.nn.sigmoid(gf) * c + jax.nn.sigmoid(gi) * jnp.tanh(gg)
        hn = jax.nn.sigmoid(go) * jnp.tanh(cn)
        return hn, cn

    h1v, c1v, h2v, c2v = h1[...], c1[...], h2[...], c2[...]
    for u in range(_UNROLL):
        xt = xw_ref[u]
        i0 = (jnp.dot(xt, pw_ref[...], preferred_element_type=jnp.float32)
              + pb_ref[...])
        h1v, c1v = cell(i0, h1v, c1v, wi1_ref[...], bi1_ref[...],
                        wh1_ref[...], bh1_ref[...])
        h2v, c2v = cell(h1v, h2v, c2v, wi2_ref[...], bi2_ref[...],
                        wh2_ref[...], bh2_ref[...])
    h1[...] = h1v
    c1[...] = c1v
    h2[...] = h2v
    c2[...] = c2v

    @pl.when(t == T // _UNROLL - 1)
    def _():
        m = jnp.mean(h2v, axis=-1, keepdims=True)
        v = jnp.mean((h2v - m) ** 2, axis=-1, keepdims=True)
        te_ref[...] = (h2v - m) / jnp.sqrt(v + 1e-5) * g_ref[...] + b_ref[...]


def _gnn1_kernel(adj_ref, te_ref, sku_ref, w1_ref, b1_ref,
                 g_ref, b_ref, w2_ref, b2_ref,
                 g1_ref, z_ref, pe_s, mw_s):
    i = pl.program_id(0)

    @pl.when(i == 0)
    def _():
        sku = sku_ref[...]
        te = te_ref[...]
        w1 = w1_ref[...]
        b1 = b1_ref[...]
        for c in range(P // _SCHUNK):
            skub = jnp.broadcast_to(sku, (_SCHUNK, B))
            prow = (c * _SCHUNK
                    + jax.lax.broadcasted_iota(jnp.int32, (_SCHUNK, B), 0))
            match = skub == prow
            lane = jax.lax.broadcasted_iota(jnp.int32, (_SCHUNK, B), 1)
            win = jnp.max(jnp.where(match, lane, -1), axis=1, keepdims=True)
            onehot = (match & (lane == win)).astype(jnp.float32)
            pe_c = jnp.dot(onehot, te, preferred_element_type=jnp.float32,
                           precision=jax.lax.Precision.HIGHEST)
            pe_s[pl.ds(c * _SCHUNK, _SCHUNK), :] = pe_c
            mw_s[pl.ds(c * _SCHUNK, _SCHUNK), :] = (
                jnp.dot(pe_c, w1, preferred_element_type=jnp.float32) + b1)

    acc = jnp.dot(adj_ref[...], mw_s[...], preferred_element_type=jnp.float32)
    u = pe_s[pl.ds(i * _PBLK, _PBLK), :] + _gelu(acc)
    m = jnp.mean(u, axis=-1, keepdims=True)
    v = jnp.mean((u - m) ** 2, axis=-1, keepdims=True)
    g1 = (u - m) / jnp.sqrt(v + 1e-5) * g_ref[...] + b_ref[...]
    g1_ref[...] = g1
    z_ref[...] = jnp.dot(g1, w2_ref[...],
                         preferred_element_type=jnp.float32) + b2_ref[...]


_NC = 2
_NS = 16
_NW = _NC * _NS
_BPW = B // _NW
_CH = 16

@functools.cache
def _sc_gathers():
    mesh = plsc.VectorSubcoreMesh(core_axis_name="c", subcore_axis_name="s",
                                  num_cores=_NC, num_subcores=_NS)

    @functools.partial(
        pl.kernel,
        out_type=jax.ShapeDtypeStruct((B, P), jnp.float32),
        mesh=mesh,
        scratch_types=[
            pltpu.VMEM((_CH,), jnp.int32),
            pltpu.VMEM((_CH, P), jnp.float32),
            pltpu.SemaphoreType.DMA,
        ],
    )
    def sc_gather_adj(sku_hbm, adj_hbm, out_hbm, idx_v, rows_v, sem):
        wid = lax.axis_index("s") * _NC + lax.axis_index("c")
        base = wid * _BPW
        for ci in range(_BPW // _CH):
            off = base + ci * _CH
            pltpu.sync_copy(sku_hbm.at[pl.ds(off, _CH)], idx_v)
            pltpu.async_copy(adj_hbm.at[idx_v], rows_v, sem).wait()
            pltpu.sync_copy(rows_v, out_hbm.at[pl.ds(off, _CH)])

    @functools.partial(
        pl.kernel,
        out_type=jax.ShapeDtypeStruct((B, H), jnp.float32),
        mesh=mesh,
        scratch_types=[
            pltpu.VMEM((_BPW,), jnp.int32),
            pltpu.VMEM((_BPW, H), jnp.float32),
            pltpu.SemaphoreType.DMA,
        ],
    )
    def sc_gather_g1(sku_hbm, g1_hbm, out_hbm, idx_v, rows_v, sem):
        wid = lax.axis_index("s") * _NC + lax.axis_index("c")
        base = wid * _BPW
        pltpu.sync_copy(sku_hbm.at[pl.ds(base, _BPW)], idx_v)
        pltpu.async_copy(g1_hbm.at[idx_v], rows_v, sem).wait()
        pltpu.sync_copy(rows_v, out_hbm.at[pl.ds(base, _BPW)])

    return sc_gather_adj, sc_gather_g1


def _head_kernel(r_ref, z_ref, g1r_ref, g_ref, b_ref, w1_ref, b1_ref,
                 w2_ref, b2_ref, out_ref):
    acc = jnp.dot(r_ref[...], z_ref[...], preferred_element_type=jnp.float32)
    u = g1r_ref[...] + _gelu(acc)
    m = jnp.mean(u, axis=-1, keepdims=True)
    v = jnp.mean((u - m) ** 2, axis=-1, keepdims=True)
    u = (u - m) / jnp.sqrt(v + 1e-5) * g_ref[...] + b_ref[...]
    hid = _gelu(jnp.dot(u, w1_ref[...], preferred_element_type=jnp.float32)
                + b1_ref[...])
    o = jnp.dot(hid, w2_ref[...], preferred_element_type=jnp.float32) + b2_ref[...]
    out_ref[...] = o


def _full(shape):
    return pl.BlockSpec(shape, lambda *_: tuple(0 for _ in shape))


@jax.jit
def kernel(x, sku_indices, adj, params):
    x450 = x.reshape(B, T * NF)
    sku2d = sku_indices.reshape(1, B).astype(jnp.int32)
    sku1d = sku_indices.astype(jnp.int32)
    sc_gather_adj, sc_gather_g1 = _sc_gathers()
    r_rows = sc_gather_adj(sku1d, adj)

    def row(v):
        return v.reshape(1, -1)

    xw = pl.pallas_call(
        _vsn_kernel,
        out_shape=jax.ShapeDtypeStruct((B, T * NF), jnp.float32),
    )(x450, params['vs_w1'], row(params['vs_b1']),
      params['vs_w2'], row(params['vs_b2']))

    xwT = xw.reshape(B, T, NF).transpose(1, 0, 2)

    lp1, lp2 = params['lstm']

    te = pl.pallas_call(
        _lstm_kernel,
        grid=(T // _UNROLL,),
        in_specs=[
            pl.BlockSpec((_UNROLL, B, NF), lambda t: (t, 0, 0)),
            _full((NF, H)), _full((1, H)),
            _full((H, 4 * H)), _full((1, 4 * H)),
            _full((H, 4 * H)), _full((1, 4 * H)),
            _full((H, 4 * H)), _full((1, 4 * H)),
            _full((H, 4 * H)), _full((1, 4 * H)),
            _full((1, H)), _full((1, H)),
        ],
        out_specs=_full((B, H)),
        out_shape=jax.ShapeDtypeStruct((B, H), jnp.float32),
        scratch_shapes=[pltpu.VMEM((B, H), jnp.float32) for _ in range(4)],
        compiler_params=pltpu.CompilerParams(
            dimension_semantics=("arbitrary",)),
    )(xwT, params['proj_w'], row(params['proj_b']),
      lp1['wih'].T, row(lp1['bih']), lp1['whh'].T, row(lp1['bhh']),
      lp2['wih'].T, row(lp2['bih']), lp2['whh'].T, row(lp2['bhh']),
      row(params['ln_g']), row(params['ln_b']))

    gnn1, gnn2 = params['gnn']
    g1, z = pl.pallas_call(
        _gnn1_kernel,
        grid=(P // _PBLK,),
        in_specs=[
            pl.BlockSpec((_PBLK, P), lambda i: (i, 0)),
            _full((B, H)), _full((1, B)),
            _full((H, H)), _full((1, H)),
            _full((1, H)), _full((1, H)),
            _full((H, H)), _full((1, H)),
        ],
        out_specs=[pl.BlockSpec((_PBLK, H), lambda i: (i, 0)),
                   pl.BlockSpec((_PBLK, H), lambda i: (i, 0))],
        out_shape=[jax.ShapeDtypeStruct((P, H), jnp.float32),
                   jax.ShapeDtypeStruct((P, H), jnp.float32)],
        scratch_shapes=[pltpu.VMEM((P, H), jnp.float32),
                        pltpu.VMEM((P, H), jnp.float32)],
        compiler_params=pltpu.CompilerParams(
            dimension_semantics=("arbitrary",)),
    )(adj, te, sku2d, gnn1['w'], row(gnn1['b']),
      row(gnn1['g']), row(gnn1['bn']),
      gnn2['w'], row(gnn2['b']))

    g1_rows = sc_gather_g1(sku1d, g1)

    out = pl.pallas_call(
        _head_kernel,
        grid=(B // _BBLK,),
        in_specs=[
            pl.BlockSpec((_BBLK, P), lambda i: (i, 0)),
            _full((P, H)),
            pl.BlockSpec((_BBLK, H), lambda i: (i, 0)),
            _full((1, H)), _full((1, H)),
            _full((H, H // 2)), _full((1, H // 2)),
            _full((H // 2, 1)), _full((1, 1)),
        ],
        out_specs=pl.BlockSpec((_BBLK, 1), lambda i: (i, 0)),
        out_shape=jax.ShapeDtypeStruct((B, 1), jnp.float32),
        compiler_params=pltpu.CompilerParams(
            dimension_semantics=("arbitrary",)),
    )(r_rows, z, g1_rows, row(gnn2['g']), row(gnn2['bn']),
      params['out_w1'], row(params['out_b1']),
      params['out_w2'], params['out_b2'].reshape(1, 1))

    return out.reshape(B)


# speedup vs baseline: 1.0479x; 1.0479x over previous
import functools

import jax
import jax.numpy as jnp
from jax import lax
from jax.experimental import pallas as pl
from jax.experimental.pallas import tpu as pltpu
from jax.experimental.pallas import tpu_sc as plsc

NF = 9
H = 128
P = 4096
B = 1024
T = 50

_PBLK = 512
_BBLK = 256
_UNROLL = 5


def _gelu(x):
    return x * (jax.lax.erf(x / 2.0 ** 0.5) + 1) / 2


def _vsn_kernel(x_ref, w1_ref, b1_ref, w2_ref, b2_ref, out_ref):
    x = x_ref[...]
    r = jax.lax.broadcasted_iota(jnp.int32, (T * NF, NF), 0)
    f = jax.lax.broadcasted_iota(jnp.int32, (T * NF, NF), 1)
    sel = (r % NF == f).astype(jnp.float32)
    xm = jnp.dot(x, sel, preferred_element_type=jnp.float32,
                 precision=jax.lax.Precision.HIGHEST) / T
    h = _gelu(jnp.dot(xm, w1_ref[...], preferred_element_type=jnp.float32)
              + b1_ref[...])
    s = jnp.dot(h, w2_ref[...], preferred_element_type=jnp.float32) + b2_ref[...]
    s = s - jnp.max(s, axis=-1, keepdims=True)
    e = jnp.exp(s)
    w = e / jnp.sum(e, axis=-1, keepdims=True)
    rT = jax.lax.broadcasted_iota(jnp.int32, (NF, T * NF), 1)
    fT = jax.lax.broadcasted_iota(jnp.int32, (NF, T * NF), 0)
    selT = (rT % NF == fT).astype(jnp.float32)
    wrep = jnp.dot(w, selT, preferred_element_type=jnp.float32,
                   precision=jax.lax.Precision.HIGHEST)
    out_ref[...] = x * wrep


def _lstm_kernel(xw_ref, pw_ref, pb_ref, wi1_ref, bi1_ref, wh1_ref, bh1_ref,
                 wi2_ref, bi2_ref, wh2_ref, bh2_ref,
                 g_ref, b_ref, te_ref, h1, c1, h2, c2):
    t = pl.program_id(0)

    @pl.when(t == 0)
    def _():
        z = jnp.zeros((B, H), jnp.float32)
        h1[...] = z
        c1[...] = z
        h2[...] = z
        c2[...] = z

    def cell(inp, h, c, wi, bi, wh, bh):
        gates = (jnp.dot(inp, wi, preferred_element_type=jnp.float32) + bi
                 + jnp.dot(h, wh, preferred_element_type=jnp.float32) + bh)
        gi = gates[:, 0 * H:1 * H]
        gf = gates[:, 1 * H:2 * H]
        gg = gates[:, 2 * H:3 * H]
        go = gates[:, 3 * H:4 * H]
        cn = jax.nn.sigmoid(gf) * c + jax.nn.sigmoid(gi) * jnp.tanh(gg)
        hn = jax.nn.sigmoid(go) * jnp.tanh(cn)
        return hn, cn

    h1v, c1v, h2v, c2v = h1[...], c1[...], h2[...], c2[...]
    for u in range(_UNROLL):
        xt = xw_ref[u]
        i0 = (jnp.dot(xt, pw_ref[...], preferred_element_type=jnp.float32)
              + pb_ref[...])
        h1v, c1v = cell(i0, h1v, c1v, wi1_ref[...], bi1_ref[...],
                        wh1_ref[...], bh1_ref[...])
        h2v, c2v = cell(h1v, h2v, c2v, wi2_ref[...], bi2_ref[...],
                        wh2_ref[...], bh2_ref[...])
    h1[...] = h1v
    c1[...] = c1v
    h2[...] = h2v
    c2[...] = c2v

    @pl.when(t == T // _UNROLL - 1)
    def _():
        m = jnp.mean(h2v, axis=-1, keepdims=True)
        v = jnp.mean((h2v - m) ** 2, axis=-1, keepdims=True)
        te_ref[...] = (h2v - m) / jnp.sqrt(v + 1e-5) * g_ref[...] + b_ref[...]


def _scatter_kernel(te_ref, sku_ref, w1_ref, b1_ref, pe_ref, mw_ref):
    blk = pl.program_id(0)
    p0 = blk * _PBLK
    sku = sku_ref[...]
    skub = jnp.broadcast_to(sku, (_PBLK, B))
    prow = p0 + jax.lax.broadcasted_iota(jnp.int32, (_PBLK, B), 0)
    match = skub == prow
    lane = jax.lax.broadcasted_iota(jnp.int32, (_PBLK, B), 1)
    win = jnp.max(jnp.where(match, lane, -1), axis=1, keepdims=True)
    onehot = (match & (lane == win)).astype(jnp.float32)
    pe = jnp.dot(onehot, te_ref[...], preferred_element_type=jnp.float32,
                 precision=jax.lax.Precision.HIGHEST)
    pe_ref[...] = pe
    mw_ref[...] = jnp.dot(pe, w1_ref[...],
                          preferred_element_type=jnp.float32) + b1_ref[...]


def _gnn1_kernel(adj_ref, mw_ref, pe_ref, g_ref, b_ref, w2_ref, b2_ref,
                 g1_ref, z_ref):
    acc = jnp.dot(adj_ref[...], mw_ref[...], preferred_element_type=jnp.float32)
    u = pe_ref[...] + _gelu(acc)
    m = jnp.mean(u, axis=-1, keepdims=True)
    v = jnp.mean((u - m) ** 2, axis=-1, keepdims=True)
    g1 = (u - m) / jnp.sqrt(v + 1e-5) * g_ref[...] + b_ref[...]
    g1_ref[...] = g1
    z_ref[...] = jnp.dot(g1, w2_ref[...],
                         preferred_element_type=jnp.float32) + b2_ref[...]


_NC = 2
_NS = 16
_NW = _NC * _NS
_BPW = B // _NW
_CH = 16

@functools.cache
def _sc_gathers():
    mesh = plsc.VectorSubcoreMesh(core_axis_name="c", subcore_axis_name="s",
                                  num_cores=_NC, num_subcores=_NS)

    @functools.partial(
        pl.kernel,
        out_type=jax.ShapeDtypeStruct((B, P), jnp.float32),
        mesh=mesh,
        scratch_types=[
            pltpu.VMEM((_CH,), jnp.int32),
            pltpu.VMEM((_CH, P), jnp.float32),
            pltpu.SemaphoreType.DMA,
        ],
    )
    def sc_gather_adj(sku_hbm, adj_hbm, out_hbm, idx_v, rows_v, sem):
        wid = lax.axis_index("s") * _NC + lax.axis_index("c")
        base = wid * _BPW
        for ci in range(_BPW // _CH):
            off = base + ci * _CH
            pltpu.sync_copy(sku_hbm.at[pl.ds(off, _CH)], idx_v)
            pltpu.async_copy(adj_hbm.at[idx_v], rows_v, sem).wait()
            pltpu.sync_copy(rows_v, out_hbm.at[pl.ds(off, _CH)])

    @functools.partial(
        pl.kernel,
        out_type=jax.ShapeDtypeStruct((B, H), jnp.float32),
        mesh=mesh,
        scratch_types=[
            pltpu.VMEM((_BPW,), jnp.int32),
            pltpu.VMEM((_BPW, H), jnp.float32),
            pltpu.SemaphoreType.DMA,
        ],
    )
    def sc_gather_g1(sku_hbm, g1_hbm, out_hbm, idx_v, rows_v, sem):
        wid = lax.axis_index("s") * _NC + lax.axis_index("c")
        base = wid * _BPW
        pltpu.sync_copy(sku_hbm.at[pl.ds(base, _BPW)], idx_v)
        pltpu.async_copy(g1_hbm.at[idx_v], rows_v, sem).wait()
        pltpu.sync_copy(rows_v, out_hbm.at[pl.ds(base, _BPW)])

    return sc_gather_adj, sc_gather_g1


def _head_kernel(r_ref, z_ref, g1r_ref, g_ref, b_ref, w1_ref, b1_ref,
                 w2_ref, b2_ref, out_ref):
    acc = jnp.dot(r_ref[...], z_ref[...], preferred_element_type=jnp.float32)
    u = g1r_ref[...] + _gelu(acc)
    m = jnp.mean(u, axis=-1, keepdims=True)
    v = jnp.mean((u - m) ** 2, axis=-1, keepdims=True)
    u = (u - m) / jnp.sqrt(v + 1e-5) * g_ref[...] + b_ref[...]
    hid = _gelu(jnp.dot(u, w1_ref[...], preferred_element_type=jnp.float32)
                + b1_ref[...])
    o = jnp.dot(hid, w2_ref[...], preferred_element_type=jnp.float32) + b2_ref[...]
    out_ref[...] = o


def _full(shape):
    return pl.BlockSpec(shape, lambda *_: tuple(0 for _ in shape))


@jax.jit
def kernel(x, sku_indices, adj, params):
    x450 = x.reshape(B, T * NF)
    sku2d = sku_indices.reshape(1, B).astype(jnp.int32)
    sku1d = sku_indices.astype(jnp.int32)
    sc_gather_adj, sc_gather_g1 = _sc_gathers()
    r_rows = sc_gather_adj(sku1d, adj)

    def row(v):
        return v.reshape(1, -1)

    xw = pl.pallas_call(
        _vsn_kernel,
        out_shape=jax.ShapeDtypeStruct((B, T * NF), jnp.float32),
    )(x450, params['vs_w1'], row(params['vs_b1']),
      params['vs_w2'], row(params['vs_b2']))

    xwT = xw.reshape(B, T, NF).transpose(1, 0, 2)

    lp1, lp2 = params['lstm']

    te = pl.pallas_call(
        _lstm_kernel,
        grid=(T // _UNROLL,),
        in_specs=[
            pl.BlockSpec((_UNROLL, B, NF), lambda t: (t, 0, 0)),
            _full((NF, H)), _full((1, H)),
            _full((H, 4 * H)), _full((1, 4 * H)),
            _full((H, 4 * H)), _full((1, 4 * H)),
            _full((H, 4 * H)), _full((1, 4 * H)),
            _full((H, 4 * H)), _full((1, 4 * H)),
            _full((1, H)), _full((1, H)),
        ],
        out_specs=_full((B, H)),
        out_shape=jax.ShapeDtypeStruct((B, H), jnp.float32),
        scratch_shapes=[pltpu.VMEM((B, H), jnp.float32) for _ in range(4)],
        compiler_params=pltpu.CompilerParams(
            dimension_semantics=("arbitrary",)),
    )(xwT, params['proj_w'], row(params['proj_b']),
      lp1['wih'].T, row(lp1['bih']), lp1['whh'].T, row(lp1['bhh']),
      lp2['wih'].T, row(lp2['bih']), lp2['whh'].T, row(lp2['bhh']),
      row(params['ln_g']), row(params['ln_b']))

    gnn1, gnn2 = params['gnn']
    pe, mw = pl.pallas_call(
        _scatter_kernel,
        grid=(P // _PBLK,),
        in_specs=[_full((B, H)), _full((1, B)),
                  _full((H, H)), _full((1, H))],
        out_specs=[pl.BlockSpec((_PBLK, H), lambda i: (i, 0)),
                   pl.BlockSpec((_PBLK, H), lambda i: (i, 0))],
        out_shape=[jax.ShapeDtypeStruct((P, H), jnp.float32),
                   jax.ShapeDtypeStruct((P, H), jnp.float32)],
        compiler_params=pltpu.CompilerParams(
            dimension_semantics=("arbitrary",)),
    )(te, sku2d, gnn1['w'], row(gnn1['b']))

    g1, z = pl.pallas_call(
        _gnn1_kernel,
        grid=(P // _PBLK,),
        in_specs=[
            pl.BlockSpec((_PBLK, P), lambda i: (i, 0)),
            _full((P, H)),
            pl.BlockSpec((_PBLK, H), lambda i: (i, 0)),
            _full((1, H)), _full((1, H)),
            _full((H, H)), _full((1, H)),
        ],
        out_specs=[pl.BlockSpec((_PBLK, H), lambda i: (i, 0)),
                   pl.BlockSpec((_PBLK, H), lambda i: (i, 0))],
        out_shape=[jax.ShapeDtypeStruct((P, H), jnp.float32),
                   jax.ShapeDtypeStruct((P, H), jnp.float32)],
        compiler_params=pltpu.CompilerParams(
            dimension_semantics=("arbitrary",)),
    )(adj, mw, pe, row(gnn1['g']), row(gnn1['bn']),
      gnn2['w'], row(gnn2['b']))

    g1_rows = sc_gather_g1(sku1d, g1)

    out = pl.pallas_call(
        _head_kernel,
        grid=(B // _BBLK,),
        in_specs=[
            pl.BlockSpec((_BBLK, P), lambda i: (i, 0)),
            _full((P, H)),
            pl.BlockSpec((_BBLK, H), lambda i: (i, 0)),
            _full((1, H)), _full((1, H)),
            _full((H, H // 2)), _full((1, H // 2)),
            _full((H // 2, 1)), _full((1, 1)),
        ],
        out_specs=pl.BlockSpec((_BBLK, 1), lambda i: (i, 0)),
        out_shape=jax.ShapeDtypeStruct((B, 1), jnp.float32),
        compiler_params=pltpu.CompilerParams(
            dimension_semantics=("arbitrary",)),
    )(r_rows, z, g1_rows, row(gnn2['g']), row(gnn2['bn']),
      params['out_w1'], row(params['out_b1']),
      params['out_w2'], params['out_b2'].reshape(1, 1))

    return out.reshape(B)
